# baseline (speedup 1.0000x reference)
import functools

import numpy as np
import jax
import jax.numpy as jnp
from jax import lax
from jax.experimental import pallas as pl
from jax.experimental.pallas import tpu as pltpu
from jax.experimental.pallas import tpu_sc as plsc

ROWS = 32
N = 1000000
TOTAL = ROWS * N
TOP_P = 0.9

_K1 = 0x28C97A78
_K2 = 0x76F86359
_KS2 = (_K1 ^ _K2 ^ 0x1BD11BDA) & 0xFFFFFFFF
_TINY = float(np.finfo(np.float32).tiny)

W = 8000
NW = N // W
VPW = W // 16
HBINS = 65536

_MESH = plsc.VectorSubcoreMesh(core_axis_name="c", subcore_axis_name="s",
                               num_cores=2)
_PARAMS = pltpu.CompilerParams(needs_layout_passes=False)


def _stats_body(x_ref, m_ref, z_ref):
    x = x_ref[...]
    m = jnp.max(x)
    z = jnp.sum(jnp.exp(x - m))
    m_ref[...] = jnp.full((1, 8, 128), m, jnp.float32)
    z_ref[...] = jnp.full((1, 8, 128), z, jnp.float32)


def _gumbel_body(o_ref):
    i = pl.program_id(0)
    shp = o_ref.shape
    blk = shp[0] * shp[1]
    r = lax.broadcasted_iota(jnp.uint32, shp, 0)
    c = lax.broadcasted_iota(jnp.uint32, shp, 1)
    p = jnp.uint32(i) * jnp.uint32(blk) + r * jnp.uint32(shp[1]) + c
    x0 = jnp.full(shp, jnp.uint32(_K1))
    x1 = p + jnp.uint32(_K2)
    ks = (_K1, _K2, _KS2)
    rots = ((13, 15, 26, 6), (17, 29, 16, 24))
    for gi in range(5):
        for rr in rots[gi % 2]:
            x0 = x0 + x1
            x1 = (x1 << jnp.uint32(rr)) | (x1 >> jnp.uint32(32 - rr))
            x1 = x1 ^ x0
        x0 = x0 + jnp.uint32(ks[(gi + 1) % 3])
        x1 = x1 + jnp.uint32((ks[(gi + 2) % 3] + gi + 1) & 0xFFFFFFFF)
    bits = x0 ^ x1
    fb = (bits >> jnp.uint32(9)) | jnp.uint32(0x3F800000)
    f = lax.bitcast_convert_type(fb, jnp.float32)
    u = (f - jnp.float32(1.0)) * jnp.float32(1.0 - _TINY) + jnp.float32(_TINY)
    u = jnp.maximum(jnp.float32(_TINY), u)
    o_ref[...] = -jnp.log(-jnp.log(u))


def _vgather(x, idx):
    dn = lax.GatherDimensionNumbers(
        offset_dims=(), collapsed_slice_dims=(0,), start_index_map=(0,))
    return lax.gather(x, idx[:, None], dn, (1,),
                      mode=lax.GatherScatterMode.PROMISE_IN_BOUNDS)


def _to_key(x_f32):
    b = plsc.bitcast(x_f32, jnp.uint32)
    neg = (b >> jnp.uint32(31)) == jnp.uint32(1)
    m = jnp.where(neg, ~b, b | jnp.uint32(0x80000000))
    return ~m


def _sort_runs(dig, lane):
    comp = (dig << 4) | lane
    sk, sv = plsc.sort_key_val(comp, lane, descending=False)
    dig_s = sk >> 4
    prev = _vgather(sk, jnp.maximum(lane - 1, 0))
    nxt = _vgather(sk, jnp.minimum(lane + 1, 15))
    is_start = (lane == 0) | (dig_s != (prev >> 4))
    is_end = (lane == 15) | (dig_s != (nxt >> 4))
    rs = plsc.cummax(jnp.where(is_start, lane, jnp.int32(0)))
    return sv, dig_s, lane - rs, is_end


def _make_pass_body(from_f32):
    def body(src, dst, cnt, win, dstb, valb, prm, sem):
        row = lax.axis_index("c") * 16 + lax.axis_index("s")
        rbase = row * N
        lane = lax.iota(jnp.int32, 16)

        def get_digit(k):
            if from_f32:
                d = _to_key(win[pl.ds(k * 16, 16)])
                return d, (d & jnp.uint32(0xFFFF)).astype(jnp.int32)
            d = win[pl.ds(k * 16, 16)]
            return d, (d >> jnp.uint32(16)).astype(jnp.int32)

        def clear_body(t, c):
            cnt[pl.ds(t * 16, 16)] = jnp.zeros((16,), jnp.int32)
            return c
        lax.fori_loop(0, HBINS // 16, clear_body, 0)

        def hist_w(wi, c):
            pltpu.sync_copy(src.at[pl.ds(rbase + wi * W, W)], win)
            def vl(k, c2):
                _, dig = get_digit(k)
                sv, dig_s, e_s, is_end = _sort_runs(dig, lane)
                plsc.addupdate_scatter(cnt, [dig_s], e_s + 1, mask=is_end)
                return c2
            return lax.fori_loop(0, VPW, vl, c)
        lax.fori_loop(0, NW, hist_w, 0)

        def pfx_body(t, carry):
            v = cnt[pl.ds(t * 16, 16)]
            s = plsc.cumsum(v)
            cnt[pl.ds(t * 16, 16)] = s - v + carry
            return carry + jnp.sum(v)
        lax.fori_loop(0, HBINS // 16, pfx_body, jnp.int32(0))

        def scat_w(wi, c):
            pltpu.sync_copy(src.at[pl.ds(rbase + wi * W, W)], win)
            def vl(k, c2):
                d, dig = get_digit(k)
                sv, dig_s, e_s, is_end = _sort_runs(dig, lane)
                base = plsc.load_gather(cnt, [dig_s])
                plsc.addupdate_scatter(cnt, [dig_s], e_s + 1, mask=is_end)
                plsc.store_scatter(prm, [sv], base + e_s)
                dstb[pl.ds(k * 16, 16)] = prm[...] + rbase
                valb[pl.ds(k * 16, 16)] = d
                return c2
            lax.fori_loop(0, VPW, vl, 0)
            pltpu.async_copy(valb, dst.at[dstb], sem).wait()
            return c
        lax.fori_loop(0, NW, scat_w, 0)

    return body


def _final_body(lg, bbuf, gg, mz, tok, cnt_unused, winf, winu, gwin,
                prm, idx16, k16, mzv, tokv, sem):
    row = lax.axis_index("c") * 16 + lax.axis_index("s")
    rbase = row * N
    lane = lax.iota(jnp.int32, 16)

    pltpu.sync_copy(mz.at[pl.ds(row * 32, 32)], mzv)
    Mv = mzv[pl.ds(0, 16)]
    Zv = mzv[pl.ds(16, 16)]

    def wl_g(wi, carry):
        pltpu.sync_copy(bbuf.at[pl.ds(rbase + wi * W, W)], winu)
        pltpu.sync_copy(gg.at[pl.ds(rbase + wi * W, W)], gwin)
        def vl(k, c2):
            cum, bs, bp = c2
            d = winu[pl.ds(k * 16, 16)]
            m = ~d
            bb = jnp.where((m >> jnp.uint32(31)) == jnp.uint32(1),
                           m & jnp.uint32(0x7FFFFFFF), ~m)
            x = plsc.bitcast(bb, jnp.float32)
            p = jnp.exp(x - Mv) / Zv
            s = plsc.cumsum(p)
            excl = cum + (s - p)
            keep = excl <= jnp.float32(TOP_P)
            g = gwin[pl.ds(k * 16, 16)]
            sc = jnp.where(keep, x + g, -jnp.inf)
            vmax = jnp.max(sc)
            posv = jnp.where(sc == vmax, wi * W + k * 16 + lane,
                             jnp.int32(2 ** 30))
            pmin = jnp.min(posv)
            better = vmax > bs
            return (cum + jnp.sum(p),
                    jnp.where(better, vmax, bs),
                    jnp.where(better, pmin, bp))
        return lax.fori_loop(0, VPW, vl, carry)

    _, _, bp = lax.fori_loop(
        0, NW, wl_g,
        (jnp.float32(0.0), jnp.float32(-jnp.inf), jnp.int32(0)))

    idx16[...] = jnp.zeros((16,), jnp.int32) + (rbase + bp)
    pltpu.async_copy(bbuf.at[idx16], k16, sem).wait()
    kst = k16[...]

    def wl_below(wi, cl):
        pltpu.sync_copy(lg.at[pl.ds(rbase + wi * W, W)], winf)
        def vl(k, c2):
            d = _to_key(winf[pl.ds(k * 16, 16)])
            return c2 + plsc.all_reduce_population_count(d < kst)
        return lax.fori_loop(0, VPW, vl, cl)
    clv = lax.fori_loop(0, NW, wl_below, jnp.zeros((16,), jnp.int32))
    mtgt = bp - jnp.max(clv)

    def wl_kth(wi, carry):
        pltpu.sync_copy(lg.at[pl.ds(rbase + wi * W, W)], winf)
        def vl(k, c2):
            ce, tk = c2
            d = _to_key(winf[pl.ds(k * 16, 16)])
            eq = d == kst
            eqi = jnp.where(eq, jnp.int32(1), jnp.int32(0))
            s = plsc.cumsum(eqi)
            pos = ce + s - eqi
            hit = eq & (pos == mtgt)
            cand = jnp.max(jnp.where(hit, wi * W + k * 16 + lane,
                                     jnp.int32(-1)))
            return (ce + jnp.sum(eqi), jnp.maximum(tk, cand))
        return lax.fori_loop(0, VPW, vl, carry)
    _, tk = lax.fori_loop(0, NW, wl_kth, (jnp.int32(0), jnp.int32(-1)))

    tokv[...] = jnp.zeros((16,), jnp.int32) + tk
    pltpu.sync_copy(tokv, tok.at[pl.ds(row * 16, 16)])


def _pass_kernel(from_f32):
    return functools.partial(
        pl.kernel,
        out_type=jax.ShapeDtypeStruct((TOTAL,), jnp.uint32),
        mesh=_MESH,
        compiler_params=_PARAMS,
        scratch_types=[
            pltpu.VMEM((HBINS,), jnp.int32),
            pltpu.VMEM((W,), jnp.float32 if from_f32 else jnp.uint32),
            pltpu.VMEM((W,), jnp.int32),
            pltpu.VMEM((W,), jnp.uint32),
            pltpu.VMEM((16,), jnp.int32),
            pltpu.SemaphoreType.DMA,
        ],
    )(_make_pass_body(from_f32))


_sc_pass1 = _pass_kernel(True)
_sc_pass2 = _pass_kernel(False)

_sc_final = functools.partial(
    pl.kernel,
    out_type=jax.ShapeDtypeStruct((ROWS * 16,), jnp.int32),
    mesh=_MESH,
    compiler_params=_PARAMS,
    scratch_types=[
        pltpu.VMEM((16,), jnp.int32),
        pltpu.VMEM((W,), jnp.float32),
        pltpu.VMEM((W,), jnp.uint32),
        pltpu.VMEM((W,), jnp.float32),
        pltpu.VMEM((16,), jnp.int32),
        pltpu.VMEM((16,), jnp.int32),
        pltpu.VMEM((16,), jnp.uint32),
        pltpu.VMEM((32,), jnp.float32),
        pltpu.VMEM((16,), jnp.int32),
        pltpu.SemaphoreType.DMA,
    ],
)(_final_body)


def kernel(logits):
    v256 = logits.reshape(256, 125000)
    m128, z128 = pl.pallas_call(
        _stats_body,
        out_shape=(jax.ShapeDtypeStruct((ROWS, 8, 128), jnp.float32),
                   jax.ShapeDtypeStruct((ROWS, 8, 128), jnp.float32)),
        grid=(ROWS,),
        in_specs=[pl.BlockSpec((8, 125000), lambda i: (i, 0))],
        out_specs=(pl.BlockSpec((1, 8, 128), lambda i: (i, 0, 0)),
                   pl.BlockSpec((1, 8, 128), lambda i: (i, 0, 0))),
    )(v256)
    g = pl.pallas_call(
        _gumbel_body,
        out_shape=jax.ShapeDtypeStruct((2000, 16000), jnp.float32),
        grid=(250,),
        out_specs=pl.BlockSpec((8, 16000), lambda i: (i, 0)),
    )()
    mzarr = jnp.stack(
        [m128[:, 0, :16], z128[:, 0, :16]], axis=1).reshape(ROWS * 32)
    flat = logits.reshape(TOTAL)
    abuf = _sc_pass1(flat)
    bbuf = _sc_pass2(abuf)
    toks = _sc_final(flat, bbuf, g.reshape(TOTAL), mzarr)
    return toks.reshape(ROWS, 16)[:, 0]
